# baseline (speedup 1.0000x reference)
import functools

import jax
import jax.numpy as jnp
from jax import lax
from jax.experimental import pallas as pl
from jax.experimental.pallas import tpu as pltpu
from jax.experimental.pallas import tpu_sc as plsc

VOCAB = 1000000
EMB = 64
B = 4096
L = 50

NC = 2
NS = 16
NW = NC * NS

N_TOT = B * L
G = 128
GROUPS = N_TOT // G
GPW = GROUPS // NW
K = 5
C = K * G
NCH = GPW // K
LANES = 16
JJ = EMB // LANES
ROWS_PER_IT = 8


def _emb_body(x_hbm, emb_hbm, pos_hbm, out_hbm, idx_v, buf, pos_v, sem):
    wid = lax.axis_index("s") * NC + lax.axis_index("c")

    pltpu.sync_copy(x_hbm.at[wid], idx_v)
    pltpu.sync_copy(pos_hbm, pos_v)
    pos_regs = [pos_v[pl.ds(j * LANES, LANES)] for j in range(JJ)]

    def chunk_body(c, carry):
        copies = [
            pltpu.async_copy(
                emb_hbm.at[idx_v.at[c * K + g]],
                buf.at[pl.ds(g * G, G)],
                sem,
            )
            for g in range(K)
        ]
        for cp in copies:
            cp.wait()

        def bias_body(i, carry2):
            r0 = i * ROWS_PER_IT
            for u in range(ROWS_PER_IT):
                for j in range(JJ):
                    sl = (r0 + u, pl.ds(j * LANES, LANES))
                    buf[sl] = buf[sl] + pos_regs[j]
            return carry2

        lax.fori_loop(0, C // ROWS_PER_IT, bias_body, 0)

        pltpu.sync_copy(buf, out_hbm.at[pl.ds(wid * (GPW * G) + c * C, C)])
        return carry

    lax.fori_loop(0, NCH, chunk_body, 0)


@jax.jit
def _emb_lookup(x2d, w_emb, w_pos):
    mesh = plsc.VectorSubcoreMesh(core_axis_name="c", subcore_axis_name="s")
    f = functools.partial(
        pl.kernel,
        mesh=mesh,
        out_type=jax.ShapeDtypeStruct((N_TOT, EMB), jnp.float32),
        scratch_types=[
            pltpu.VMEM((GPW, G), jnp.int32),
            pltpu.VMEM((C, EMB), jnp.float32),
            pltpu.VMEM((EMB,), jnp.float32),
            pltpu.SemaphoreType.DMA,
        ],
        compiler_params=pltpu.CompilerParams(use_tc_tiling_on_sc=False),
    )(_emb_body)
    return f(x2d, w_emb, w_pos)


def kernel(x, W_emb, W_pos):
    x2d = jnp.reshape(x.astype(jnp.int32), (NW, GPW, G))
    out = _emb_lookup(x2d, W_emb, W_pos)
    return jnp.reshape(out, (B, L, EMB))


# speedup vs baseline: 1.0267x; 1.0267x over previous
import functools

import jax
import jax.numpy as jnp
from jax import lax
from jax.experimental import pallas as pl
from jax.experimental.pallas import tpu as pltpu
from jax.experimental.pallas import tpu_sc as plsc

VOCAB = 1000000
EMB = 64
B = 4096
L = 50

NC = 2
NS = 16
NW = NC * NS

N_TOT = B * L
G = 128
GROUPS = N_TOT // G
GPW = GROUPS // NW
R = 5
LANES = 16
JJ = EMB // LANES
ROWS_PER_IT = 8


def _emb_body(x_hbm, emb_hbm, pos_hbm, out_hbm, idx_v, buf, pos_v, *sems):
    gsems, osems = sems[:R], sems[R:]
    wid = lax.axis_index("s") * NC + lax.axis_index("c")
    out_base = wid * (GPW * G)

    pltpu.sync_copy(x_hbm.at[wid], idx_v)
    pltpu.sync_copy(pos_hbm, pos_v)
    pos_regs = [pos_v[pl.ds(j * LANES, LANES)] for j in range(JJ)]

    def fire(g, s):
        pltpu.async_copy(emb_hbm.at[idx_v.at[g]], buf.at[s], gsems[s])

    def drain(s, sem):
        pltpu.make_async_copy(emb_hbm.at[pl.ds(0, G)], buf.at[s], sem).wait()

    for g in range(R):
        fire(g, g)

    def outer(it, carry):
        for s in range(R):
            g = it * R + s
            s2 = (s + R - 2) % R

            @pl.when(g >= 2)
            def _():
                drain(s2, osems[s2])

            @pl.when((g >= 2) & (g + 3 < GPW))
            def _():
                fire(g + 3, s2)

            drain(s, gsems[s])

            def bias_body(i, c2):
                r0 = i * ROWS_PER_IT
                for u in range(ROWS_PER_IT):
                    for j in range(JJ):
                        v = buf[s, r0 + u, pl.ds(j * LANES, LANES)]
                        buf[s, r0 + u, pl.ds(j * LANES, LANES)] = v + pos_regs[j]
                return c2

            lax.fori_loop(0, G // ROWS_PER_IT, bias_body, 0)

            pltpu.async_copy(
                buf.at[s], out_hbm.at[pl.ds(out_base + g * G, G)], osems[s]
            )
        return carry

    lax.fori_loop(0, GPW // R, outer, 0)

    drain((GPW - 2) % R, osems[(GPW - 2) % R])
    drain((GPW - 1) % R, osems[(GPW - 1) % R])


@jax.jit
def _emb_lookup(x3d, w_emb, w_pos):
    mesh = plsc.VectorSubcoreMesh(core_axis_name="c", subcore_axis_name="s")
    f = functools.partial(
        pl.kernel,
        mesh=mesh,
        out_type=jax.ShapeDtypeStruct((N_TOT, EMB), jnp.float32),
        scratch_types=[
            pltpu.VMEM((GPW, G), jnp.int32),
            pltpu.VMEM((R, G, EMB), jnp.float32),
            pltpu.VMEM((EMB,), jnp.float32),
        ]
        + [pltpu.SemaphoreType.DMA] * (2 * R),
        compiler_params=pltpu.CompilerParams(use_tc_tiling_on_sc=False),
    )(_emb_body)
    return f(x3d, w_emb, w_pos)


def kernel(x, W_emb, W_pos):
    x3d = jnp.reshape(x.astype(jnp.int32), (NW, GPW, G))
    out = _emb_lookup(x3d, W_emb, W_pos)
    return jnp.reshape(out, (B, L, EMB))


# speedup vs baseline: 1.1393x; 1.1096x over previous
import functools

import jax
import jax.numpy as jnp
from jax import lax
from jax.experimental import pallas as pl
from jax.experimental.pallas import tpu as pltpu
from jax.experimental.pallas import tpu_sc as plsc

VOCAB = 1000000
EMB = 64
PADW = 128
B = 4096
L = 50

NC = 2
NS = 16
NW = NC * NS

BPW = B // NW
R = 4
RO = 2
LANES = 16
JJ = EMB // LANES


def _emb_body(x_hbm, emb_hbm, pos_hbm, out_hbm, *refs):
    idx_v, pos_v = refs[0], refs[1]
    gslots = refs[2 : 2 + R]
    oslots = refs[2 + R : 2 + R + RO]
    sems = refs[2 + R + RO :]
    gsems, osems = sems[:R], sems[R:]

    wid = lax.axis_index("s") * NC + lax.axis_index("c")
    b_base = wid * BPW

    pltpu.sync_copy(x_hbm.at[wid], idx_v)
    pltpu.sync_copy(pos_hbm, pos_v)
    pos_regs = [pos_v[pl.ds(j * LANES, LANES)] for j in range(JJ)]

    def fire(b, q):
        pltpu.async_copy(emb_hbm.at[idx_v.at[b]], gslots[q], gsems[q])

    def wait_gather(b, q):
        pltpu.make_async_copy(
            emb_hbm.at[idx_v.at[b]], gslots[q], gsems[q]
        ).wait()

    def wait_out(p, b):
        pltpu.make_async_copy(oslots[p], out_hbm.at[b], osems[p]).wait()

    for q in range(R):
        fire(q, q)

    def do_batch(b, q, p):
        @pl.when(b >= RO)
        def _():
            wait_out(p, b_base + b - RO)

        wait_gather(b, q)

        for r in range(L):
            for j in range(JJ):
                v = gslots[q][r, pl.ds(j * LANES, LANES)]
                oslots[p][r, pl.ds(j * LANES, LANES)] = v + pos_regs[j]

        @pl.when(b < BPW - R)
        def _():
            fire(b + R, q)

        pltpu.async_copy(oslots[p], out_hbm.at[b_base + b], osems[p])

    def outer(bb, carry):
        for q in range(R):
            do_batch(bb * R + q, q, q % RO)
        return carry

    lax.fori_loop(0, BPW // R, outer, 0)

    wait_out(0, b_base + BPW - 2)
    wait_out(1, b_base + BPW - 1)


@jax.jit
def _emb_lookup(x3, embp, w_pos):
    mesh = plsc.VectorSubcoreMesh(core_axis_name="c", subcore_axis_name="s")
    f = functools.partial(
        pl.kernel,
        mesh=mesh,
        out_type=jax.ShapeDtypeStruct((B, L, EMB), jnp.float32),
        scratch_types=[
            pltpu.VMEM((BPW, L), jnp.int32),
            pltpu.VMEM((EMB,), jnp.float32),
        ]
        + [pltpu.VMEM((L, PADW), jnp.float32) for _ in range(R)]
        + [pltpu.VMEM((L, EMB), jnp.float32) for _ in range(RO)]
        + [pltpu.SemaphoreType.DMA] * (R + RO),
        compiler_params=pltpu.CompilerParams(
            use_tc_tiling_on_sc=True, needs_layout_passes=False
        ),
    )(_emb_body)
    return f(x3, embp, w_pos)


def kernel(x, W_emb, W_pos):
    x3 = jnp.reshape(x.astype(jnp.int32), (NW, BPW, L))
    embp = jnp.pad(W_emb, ((0, 0), (0, PADW - EMB)))
    return _emb_lookup(x3, embp, W_pos)
